# baseline (speedup 1.0000x reference)
import functools

import jax
import jax.numpy as jnp
from jax.experimental import pallas as pl

DIM = 192
WS = 8
RATIO = 0.5


def _conv2d(x, w, b=None, stride=1, padding=0, groups=1, dilation=1):
    out = jax.lax.conv_general_dilated(
        x, w, (stride, stride), [(padding, padding), (padding, padding)],
        rhs_dilation=(dilation, dilation), feature_group_count=groups,
        dimension_numbers=('NCHW', 'OIHW', 'NCHW'))
    if b is not None:
        out = out + b[None, :, None, None]
    return out


def _layernorm_cf(x, w, b, eps=1e-6):
    u = jnp.mean(x, axis=1, keepdims=True)
    s = jnp.mean((x - u) ** 2, axis=1, keepdims=True)
    xn = (x - u) / jnp.sqrt(s + eps)
    return w[None, :, None, None] * xn + b[None, :, None, None]


def _predictor(p, cond):
    x = _conv2d(cond, p['p_in_w'], p['p_in_b'])
    x = _layernorm_cf(x, p['p_ln_w'], p['p_ln_b'])
    x = jax.nn.leaky_relu(x, 0.1)
    off = jax.nn.leaky_relu(_conv2d(x, p['p_off1_w'], p['p_off1_b']), 0.1)
    offsets = jnp.tanh(_conv2d(off, p['p_off2_w'], p['p_off2_b'])) * 8.0
    x3 = _conv2d(x, p['p_conv_w'], p['p_conv_b'])
    x1, x2 = jnp.split(x3, 2, axis=1)
    ca = jax.nn.sigmoid(_conv2d(jnp.mean(x1, axis=(2, 3), keepdims=True), p['p_ca_w'], p['p_ca_b']))
    sa = jax.nn.sigmoid(_conv2d(x2, p['p_sa_w'], p['p_sa_b'], padding=1))
    xm = jnp.mean(x, axis=1, keepdims=True)
    B, _, H, W = xm.shape
    h, w = H // WS, W // WS
    xm = xm.reshape(B, 1, h, WS, w, WS).transpose(0, 3, 5, 1, 2, 4).reshape(B, WS * WS, h, w)
    ps = _conv2d(xm, p['p_mask1_w'], p['p_mask1_b'])
    ps = _conv2d(ps, p['p_mask2_w'], p['p_mask2_b'])
    ps = jax.nn.softmax(ps, axis=-1)
    pred_score = ps.reshape(B, h * w, 2)
    return offsets, ca, sa, pred_score


def _conv2d_cl(x, w, b=None, padding=0, groups=1, dilation=1):
    wx = jnp.transpose(w, (2, 3, 1, 0))
    out = jax.lax.conv_general_dilated(
        x, wx, (1, 1), [(padding, padding), (padding, padding)],
        rhs_dilation=(dilation, dilation), feature_group_count=groups,
        dimension_numbers=('NHWC', 'HWIO', 'NHWC'))
    if b is not None:
        out = out + b[None, None, None, :]
    return out


def _flow_warp_cl(x, flow):
    n, h, w, c = x.shape
    gy, gx = jnp.meshgrid(jnp.arange(h, dtype=x.dtype), jnp.arange(w, dtype=x.dtype), indexing='ij')
    vx = gx[None] + flow[..., 0]
    vy = gy[None] + flow[..., 1]
    x0 = jnp.floor(vx); y0 = jnp.floor(vy)
    wx = (vx - x0)[..., None]; wy = (vy - y0)[..., None]
    bidx = jnp.arange(n)[:, None, None]
    def gat(yi, xi):
        yi = jnp.clip(yi, 0, h - 1).astype(jnp.int32)
        xi = jnp.clip(xi, 0, w - 1).astype(jnp.int32)
        return x[bidx, yi, xi]
    v00 = gat(y0, x0); v01 = gat(y0, x0 + 1)
    v10 = gat(y0 + 1, x0); v11 = gat(y0 + 1, x0 + 1)
    return v00 * (1 - wx) * (1 - wy) + v01 * wx * (1 - wy) + v10 * (1 - wx) * wy + v11 * wx * wy


def _keep_mask(pred_score, Nw):
    key = jax.random.key(42)
    g = jax.random.gumbel(key, pred_score.shape, dtype=pred_score.dtype)
    y = jax.nn.softmax(pred_score + g, axis=2)
    mask = (y[:, :, 0:1] >= y[:, :, 1:2]).astype(pred_score.dtype)
    r = jnp.mean(mask).astype(jnp.float32)
    scale = jnp.float32(Nw * 2 * RATIO)
    rb = jax.lax.bitcast_convert_type(r, jnp.uint32)
    rh = jax.lax.bitcast_convert_type(rb & jnp.uint32(0xFFFFF000), jnp.float32)
    rl = r - rh
    a_hi = scale * rh
    a_lo = scale * rl
    fa = jnp.floor(a_hi)
    frac = a_hi - fa
    carry = ((frac - jnp.float32(1.0)) + a_lo) >= 0
    nk = fa + jnp.where(carry, jnp.float32(1.0), jnp.float32(0.0))
    num_keep = jnp.minimum(nk, jnp.float32(Nw)).astype(jnp.int32)
    idx = jnp.argsort(-pred_score[:, :, 0], axis=1)
    rank = jnp.argsort(idx, axis=1)
    keep = rank < num_keep
    return keep


def _mm_kernel(x_ref, w_ref, b_ref, o_ref):
    o_ref[...] = jnp.dot(x_ref[...], w_ref[...],
                         preferred_element_type=jnp.float32) + b_ref[...]


def _mm(x, w, b, bm=2048):
    M, K = x.shape
    Co = w.shape[1]
    assert M % bm == 0, (M, bm)
    return pl.pallas_call(
        _mm_kernel,
        grid=(M // bm,),
        in_specs=[pl.BlockSpec((bm, K), lambda i: (i, 0)),
                  pl.BlockSpec((K, Co), lambda i: (0, 0)),
                  pl.BlockSpec((1, Co), lambda i: (0, 0))],
        out_specs=pl.BlockSpec((bm, Co), lambda i: (i, 0)),
        out_shape=jax.ShapeDtypeStruct((M, Co), jnp.float32),
    )(x, w, b.reshape(1, Co))


def _attn_kernel(nwin, x_ref, k_ref, v_ref, sa_ref, ca_ref, sel_ref,
                 wq_ref, bq_ref, wk_ref, bk_ref, o_ref):
    C = x_ref.shape[-1]
    ws = WS
    xb = x_ref[0]
    kb = k_ref[0]
    vb = v_ref[0]
    W = xb.shape[1]
    x2 = xb.reshape(ws * W, C)
    k2 = kb.reshape(ws * W, C)
    q = jnp.dot(x2, wq_ref[...], preferred_element_type=jnp.float32) + bq_ref[...]
    kk = jnp.dot(k2, wk_ref[...], preferred_element_type=jnp.float32) + bk_ref[...]
    q = q.reshape(ws, W, C)
    kk = kk.reshape(ws, W, C)
    vs = vb * sa_ref[0][..., None] * ca_ref[0, 0][None, None, :]
    for wi in range(nwin):
        sl = slice(wi * ws, (wi + 1) * ws)
        qw = q[:, sl, :].reshape(ws * ws, C)
        kw = kk[:, sl, :].reshape(ws * ws, C)
        vw = vb[:, sl, :].reshape(ws * ws, C)
        aw = jax.lax.dot_general(qw, kw, (((1,), (1,)), ((), ())),
                                 preferred_element_type=jnp.float32)
        aw = jax.nn.softmax(aw, axis=-1)
        fw = jnp.dot(aw, vw, preferred_element_type=jnp.float32)
        vsw = vs[:, sl, :].reshape(ws * ws, C)
        sel = sel_ref[0, 0, wi]
        ow = jnp.where(sel > 0, fw, vsw)
        o_ref[0, :, sl, :] = ow.reshape(ws, ws, C)


def _window_attention(x_cl, k_cl, v_cl, sa, ca, keepf, wqT, bq, wkT, bk):
    N, H, W, C = x_cl.shape
    h, w = H // WS, W // WS
    kern = functools.partial(_attn_kernel, w)
    return pl.pallas_call(
        kern,
        grid=(N, h),
        in_specs=[
            pl.BlockSpec((1, WS, W, C), lambda n, i: (n, i, 0, 0)),
            pl.BlockSpec((1, WS, W, C), lambda n, i: (n, i, 0, 0)),
            pl.BlockSpec((1, WS, W, C), lambda n, i: (n, i, 0, 0)),
            pl.BlockSpec((1, WS, W), lambda n, i: (n, i, 0)),
            pl.BlockSpec((1, 1, C), lambda n, i: (n, 0, 0)),
            pl.BlockSpec((1, 1, w), lambda n, i: (n * (H // WS) + i, 0, 0)),
            pl.BlockSpec((C, C), lambda n, i: (0, 0)),
            pl.BlockSpec((1, C), lambda n, i: (0, 0)),
            pl.BlockSpec((C, C), lambda n, i: (0, 0)),
            pl.BlockSpec((1, C), lambda n, i: (0, 0)),
        ],
        out_specs=pl.BlockSpec((1, WS, W, C), lambda n, i: (n, i, 0, 0)),
        out_shape=jax.ShapeDtypeStruct((N, H, W, C), jnp.float32),
    )(x_cl, k_cl, v_cl, sa, ca.reshape(N, 1, C), keepf, wqT, bq.reshape(1, C),
      wkT, bk.reshape(1, C))


def kernel(x, condition_global, params):
    p = params
    N, C, H, W = x.shape
    h, w = H // WS, W // WS
    Nw = h * w

    x_cl = jnp.transpose(x, (0, 2, 3, 1))
    x_flat = x_cl.reshape(N * H * W, C)

    wv = p['Wv'][:, :, 0, 0]
    v_flat = _mm(x_flat, wv.T, p['bv'])
    v_cl = v_flat.reshape(N, H, W, C)

    ls = jnp.linspace(-1.0, 1.0, WS)
    ga, gb = jnp.meshgrid(ls, ls, indexing='ij')
    cw = jnp.tile(jnp.stack([ga, gb])[None], (N, 1, h, w)).astype(x.dtype)
    v = jnp.transpose(v_cl, (0, 3, 1, 2))
    cond = jnp.concatenate([v, condition_global, cw], axis=1)
    offsets, ca, sa, pred_score = _predictor(p, cond)

    keep = _keep_mask(pred_score, Nw)

    k_cl = x_cl + _flow_warp_cl(x_cl, jnp.transpose(offsets, (0, 2, 3, 1)))

    sa2 = sa[:, 0, :, :]
    ca2 = ca[:, :, 0, 0]
    keepf = keep.astype(jnp.float32).reshape(N * h, 1, w)

    out_cl = _window_attention(x_cl, k_cl, v_cl, sa2, ca2, keepf,
                               p['Wq'].T, p['bq'], p['Wk'].T, p['bk'])

    cs = _conv2d_cl(out_cl, p['Wcs1'], p['bcs1'], padding=1, groups=C)
    cs = _conv2d_cl(cs, p['Wcs2'], p['bcs2'], padding=2, groups=C, dilation=2)
    out2 = jax.nn.gelu(cs, approximate=False) * ca2[:, None, None, :] + out_cl

    wo = p['Wout'][:, :, 0, 0]
    res = _mm(out2.reshape(N * H * W, C), wo.T, p['bout'])
    return jnp.transpose(res.reshape(N, H, W, C), (0, 3, 1, 2))


# speedup vs baseline: 1.0927x; 1.0927x over previous
import functools

import jax
import jax.numpy as jnp
from jax.experimental import pallas as pl

DIM = 192
WS = 8
RATIO = 0.5


def _conv2d(x, w, b=None, stride=1, padding=0, groups=1, dilation=1):
    out = jax.lax.conv_general_dilated(
        x, w, (stride, stride), [(padding, padding), (padding, padding)],
        rhs_dilation=(dilation, dilation), feature_group_count=groups,
        dimension_numbers=('NCHW', 'OIHW', 'NCHW'))
    if b is not None:
        out = out + b[None, :, None, None]
    return out


def _layernorm_cf(x, w, b, eps=1e-6):
    u = jnp.mean(x, axis=1, keepdims=True)
    s = jnp.mean((x - u) ** 2, axis=1, keepdims=True)
    xn = (x - u) / jnp.sqrt(s + eps)
    return w[None, :, None, None] * xn + b[None, :, None, None]


def _flow_warp(x, flow):
    n, c, h, w = x.shape
    gy, gx = jnp.meshgrid(jnp.arange(h, dtype=x.dtype), jnp.arange(w, dtype=x.dtype), indexing='ij')
    vx = gx[None] + flow[..., 0]
    vy = gy[None] + flow[..., 1]
    x0 = jnp.floor(vx); y0 = jnp.floor(vy)
    wx = (vx - x0)[..., None]; wy = (vy - y0)[..., None]
    xh = jnp.transpose(x, (0, 2, 3, 1))
    bidx = jnp.arange(n)[:, None, None]
    def gat(yi, xi):
        yi = jnp.clip(yi, 0, h - 1).astype(jnp.int32)
        xi = jnp.clip(xi, 0, w - 1).astype(jnp.int32)
        return xh[bidx, yi, xi]
    v00 = gat(y0, x0); v01 = gat(y0, x0 + 1)
    v10 = gat(y0 + 1, x0); v11 = gat(y0 + 1, x0 + 1)
    out = v00 * (1 - wx) * (1 - wy) + v01 * wx * (1 - wy) + v10 * (1 - wx) * wy + v11 * wx * wy
    return jnp.transpose(out, (0, 3, 1, 2))


def _predictor(p, cond):
    x = _conv2d(cond, p['p_in_w'], p['p_in_b'])
    x = _layernorm_cf(x, p['p_ln_w'], p['p_ln_b'])
    x = jax.nn.leaky_relu(x, 0.1)
    off = jax.nn.leaky_relu(_conv2d(x, p['p_off1_w'], p['p_off1_b']), 0.1)
    offsets = jnp.tanh(_conv2d(off, p['p_off2_w'], p['p_off2_b'])) * 8.0
    x3 = _conv2d(x, p['p_conv_w'], p['p_conv_b'])
    x1, x2 = jnp.split(x3, 2, axis=1)
    ca = jax.nn.sigmoid(_conv2d(jnp.mean(x1, axis=(2, 3), keepdims=True), p['p_ca_w'], p['p_ca_b']))
    sa = jax.nn.sigmoid(_conv2d(x2, p['p_sa_w'], p['p_sa_b'], padding=1))
    xm = jnp.mean(x, axis=1, keepdims=True)
    B, _, H, W = xm.shape
    h, w = H // WS, W // WS
    xm = xm.reshape(B, 1, h, WS, w, WS).transpose(0, 3, 5, 1, 2, 4).reshape(B, WS * WS, h, w)
    ps = _conv2d(xm, p['p_mask1_w'], p['p_mask1_b'])
    ps = _conv2d(ps, p['p_mask2_w'], p['p_mask2_b'])
    ps = jax.nn.softmax(ps, axis=-1)
    pred_score = ps.reshape(B, h * w, 2)
    return offsets, ca, sa, pred_score


def _keep_mask(pred_score, Nw):
    key = jax.random.key(42)
    g = jax.random.gumbel(key, pred_score.shape, dtype=pred_score.dtype)
    y = jax.nn.softmax(pred_score + g, axis=2)
    mask = (y[:, :, 0:1] >= y[:, :, 1:2]).astype(pred_score.dtype)
    r = jnp.mean(mask).astype(jnp.float32)
    scale = jnp.float32(Nw * 2 * RATIO)
    rb = jax.lax.bitcast_convert_type(r, jnp.uint32)
    rh = jax.lax.bitcast_convert_type(rb & jnp.uint32(0xFFFFF000), jnp.float32)
    rl = r - rh
    a_hi = scale * rh
    a_lo = scale * rl
    fa = jnp.floor(a_hi)
    frac = a_hi - fa
    carry = ((frac - jnp.float32(1.0)) + a_lo) >= 0
    nk = fa + jnp.where(carry, jnp.float32(1.0), jnp.float32(0.0))
    num_keep = jnp.minimum(nk, jnp.float32(Nw)).astype(jnp.int32)
    idx = jnp.argsort(-pred_score[:, :, 0], axis=1)
    rank = jnp.argsort(idx, axis=1)
    keep = rank < num_keep
    return keep


def _attn_kernel(nwin, x_ref, k_ref, v_ref, wq_ref, bq_ref, wk_ref, bk_ref,
                 o_ref):
    C = x_ref.shape[-1]
    ws = WS
    xb = x_ref[0]
    kb = k_ref[0]
    vb = v_ref[0]
    W = xb.shape[1]
    x2 = xb.reshape(ws * W, C)
    k2 = kb.reshape(ws * W, C)
    q = jnp.dot(x2, wq_ref[...], preferred_element_type=jnp.float32) + bq_ref[...]
    kk = jnp.dot(k2, wk_ref[...], preferred_element_type=jnp.float32) + bk_ref[...]
    q = q.reshape(ws, W, C)
    kk = kk.reshape(ws, W, C)
    for wi in range(nwin):
        sl = slice(wi * ws, (wi + 1) * ws)
        qw = q[:, sl, :].reshape(ws * ws, C)
        kw = kk[:, sl, :].reshape(ws * ws, C)
        vw = vb[:, sl, :].reshape(ws * ws, C)
        aw = jax.lax.dot_general(qw, kw, (((1,), (1,)), ((), ())),
                                 preferred_element_type=jnp.float32)
        aw = jax.nn.softmax(aw, axis=-1)
        fw = jnp.dot(aw, vw, preferred_element_type=jnp.float32)
        o_ref[0, :, sl, :] = fw.reshape(ws, ws, C)


def _window_attention(x_cl, k_cl, v_cl, wqT, bq, wkT, bk):
    N, H, W, C = x_cl.shape
    h = H // WS
    kern = functools.partial(_attn_kernel, W // WS)
    return pl.pallas_call(
        kern,
        grid=(N, h),
        in_specs=[
            pl.BlockSpec((1, WS, W, C), lambda n, i: (n, i, 0, 0)),
            pl.BlockSpec((1, WS, W, C), lambda n, i: (n, i, 0, 0)),
            pl.BlockSpec((1, WS, W, C), lambda n, i: (n, i, 0, 0)),
            pl.BlockSpec((C, C), lambda n, i: (0, 0)),
            pl.BlockSpec((1, C), lambda n, i: (0, 0)),
            pl.BlockSpec((C, C), lambda n, i: (0, 0)),
            pl.BlockSpec((1, C), lambda n, i: (0, 0)),
        ],
        out_specs=pl.BlockSpec((1, WS, W, C), lambda n, i: (n, i, 0, 0)),
        out_shape=jax.ShapeDtypeStruct((N, H, W, C), jnp.float32),
    )(x_cl, k_cl, v_cl, wqT, bq.reshape(1, C), wkT, bk.reshape(1, C))


def kernel(x, condition_global, params):
    p = params
    N, C, H, W = x.shape
    h, w = H // WS, W // WS
    Nw = h * w

    v = _conv2d(x, p['Wv'], p['bv'])

    ls = jnp.linspace(-1.0, 1.0, WS)
    ga, gb = jnp.meshgrid(ls, ls, indexing='ij')
    cw = jnp.tile(jnp.stack([ga, gb])[None], (N, 1, h, w)).astype(x.dtype)
    cond = jnp.concatenate([v, condition_global, cw], axis=1)
    offsets, ca, sa, pred_score = _predictor(p, cond)

    keep = _keep_mask(pred_score, Nw)

    k = x + _flow_warp(x, jnp.transpose(offsets, (0, 2, 3, 1)))

    x_cl = jnp.transpose(x, (0, 2, 3, 1))
    k_cl = jnp.transpose(k, (0, 2, 3, 1))
    v_cl = jnp.transpose(v, (0, 2, 3, 1))
    f_cl = _window_attention(x_cl, k_cl, v_cl,
                             p['Wq'].T, p['bq'], p['Wk'].T, p['bk'])
    f_attn = f_cl.reshape(N, h, WS, w, WS, C).transpose(0, 1, 3, 2, 4, 5) \
                 .reshape(N, Nw, WS * WS * C)

    sca = sa * ca
    vs = v * sca
    def to_win(t):
        B, Cc = t.shape[0], t.shape[1]
        return t.reshape(B, Cc, h, WS, w, WS).transpose(0, 2, 4, 3, 5, 1).reshape(B, h * w, WS * WS * Cc)
    vsw = to_win(vs)
    out_w = jnp.where(keep[..., None], f_attn, vsw)
    out = out_w.reshape(N, h, w, WS, WS, C).transpose(0, 5, 1, 3, 2, 4).reshape(N, C, H, W)
    cs = _conv2d(out, p['Wcs1'], p['bcs1'], padding=1, groups=C)
    cs = _conv2d(cs, p['Wcs2'], p['bcs2'], padding=2, groups=C, dilation=2)
    out = jax.nn.gelu(cs, approximate=False) * ca + out
    out = _conv2d(out, p['Wout'], p['bout'])
    return out
